# Initial kernel scaffold; baseline (speedup 1.0000x reference)
import jax
import jax.numpy as jnp
from jax.experimental import pallas as pl


def kernel(scores, decode_bboxes, distances, anchors, gt_labels, gt_bboxes, gt_distances, gt_mask):
    raise NotImplementedError("write your pallas kernel here")


# speedup vs baseline: 39.7918x; 39.7918x over previous
import numpy as np
import jax
import jax.numpy as jnp
from jax.experimental import pallas as pl

_NUM_CLASSES = 80
_K = 10
_EPSILON = 1e-09


def _atan_pos(x):
    t38 = 2.414213562373095
    t8 = 0.4142135623730950
    use_big = x > t38
    use_mid = (x > t8) & (~use_big)
    arg = jnp.where(use_big, -1.0 / x, jnp.where(use_mid, (x - 1.0) / (x + 1.0), x))
    base = jnp.where(use_big, jnp.float32(np.pi / 2),
                     jnp.where(use_mid, jnp.float32(np.pi / 4), jnp.float32(0.0)))
    z = arg * arg
    poly = (((8.05374449538e-2 * z - 1.38776856032e-1) * z
             + 1.99777106478e-1) * z - 3.33329491539e-1) * z * arg + arg
    return base + poly


def _encoder_kernel(scores_t_ref, decode_t_ref, anchors_t_ref, gt_labels_ref,
                    gt_bboxes_ref, gt_dist_ref, gt_mask_ref,
                    bbox_out_ref, cls_out_ref, dist_out_ref, fg_out_ref):
    f32 = jnp.float32
    C, A = scores_t_ref.shape[1], scores_t_ref.shape[2]
    G = gt_bboxes_ref.shape[1]

    scores_t = scores_t_ref[0]
    dec = decode_t_ref[0]
    anc = anchors_t_ref[...]
    gtb = gt_bboxes_ref[0]
    labels = gt_labels_ref[0]
    gmask = gt_mask_ref[0]
    gdist = gt_dist_ref[0]

    d_x1 = dec[0:1, :]
    d_y1 = dec[1:2, :]
    d_x2 = dec[2:3, :]
    d_y2 = dec[3:4, :]
    a_x = anc[0:1, :]
    a_y = anc[1:2, :]
    g_x1 = gtb[:, 0:1]
    g_y1 = gtb[:, 1:2]
    g_x2 = gtb[:, 2:3]
    g_y2 = gtb[:, 3:4]

    cls = jnp.maximum(labels, 0)
    onehot_cls = (jax.lax.broadcasted_iota(jnp.int32, (G, C), 1) == cls).astype(f32)
    bbox_scores = jax.lax.dot_general(
        onehot_cls, scores_t, (((1,), (0,)), ((), ())),
        preferred_element_type=f32, precision=jax.lax.Precision.HIGHEST)

    eps = 1e-9
    x1 = jnp.maximum(g_x1, d_x1)
    y1 = jnp.maximum(g_y1, d_y1)
    x2 = jnp.minimum(g_x2, d_x2)
    y2 = jnp.minimum(g_y2, d_y2)
    inter = jnp.maximum(x2 - x1, 0.0) * jnp.maximum(y2 - y1, 0.0)
    w1 = g_x2 - g_x1
    h1 = g_y2 - g_y1
    w2 = d_x2 - d_x1
    h2 = d_y2 - d_y1
    union = w1 * h1 + w2 * h2 - inter
    iou = inter / (union + eps)
    cw = jnp.maximum(g_x2, d_x2) - jnp.minimum(g_x1, d_x1)
    ch = jnp.maximum(g_y2, d_y2) - jnp.minimum(g_y1, d_y1)
    c2 = cw * cw + ch * ch + eps
    dx = d_x1 + d_x2 - g_x1 - g_x2
    dy = d_y1 + d_y2 - g_y1 - g_y2
    rho2 = (dx * dx + dy * dy) * 0.25
    atan_g = _atan_pos(w1 / (h1 + eps))
    atan_d = _atan_pos(w2 / (h2 + eps))
    dv = atan_d - atan_g
    v = (4.0 / (np.pi ** 2)) * (dv * dv)
    alpha_t = v / (v - iou + 1.0 + eps)
    ciou = iou - (rho2 / c2 + v * alpha_t)

    o2 = ciou * ciou
    o6 = o2 * o2 * o2
    inbox = (g_x1 < a_x) & (g_y1 < a_y) & (g_x2 > a_x) & (g_y2 > a_y)
    keep = inbox & (gmask > 0.0)
    align = jnp.where(keep, jnp.sqrt(bbox_scores) * o6, 0.0)

    iota_a = jax.lax.broadcasted_iota(jnp.int32, (G, A), 1)
    work = align
    match = jnp.zeros((G, A), f32)
    for _ in range(_K):
        m = jnp.max(work, axis=1, keepdims=True)
        idx = jnp.min(jnp.where(work == m, iota_a, A), axis=1, keepdims=True)
        sel = (iota_a == idx) & (m > 0.0)
        match = jnp.where(sel, 1.0, match)
        work = jnp.where(sel, -1.0, work)

    ov_m = ciou * match
    al_m = align * match

    best_ov = jnp.max(ov_m, axis=0, keepdims=True)
    iota_g = jax.lax.broadcasted_iota(jnp.int32, (G, A), 0)
    best_g = jnp.min(jnp.where(ov_m == best_ov, iota_g, G), axis=0, keepdims=True)
    matched = best_ov > 0.0
    onehot_a = iota_g == best_g

    def gather(col):
        return jnp.sum(jnp.where(onehot_a, col, 0.0), axis=0, keepdims=True)

    gb_x1 = gather(g_x1)
    gb_y1 = gather(g_y1)
    gb_x2 = gather(g_x2)
    gb_y2 = gather(g_y2)
    g_lab = gather(labels.astype(f32))
    g_dst = gather(gdist)

    max_al = jnp.max(al_m, axis=1, keepdims=True)
    max_ov = jnp.max(ov_m, axis=1, keepdims=True)
    ratio = max_ov / (max_al + _EPSILON)
    norm = jnp.max(al_m * ratio, axis=0, keepdims=True)

    neg1 = jnp.float32(-1.0)
    bbox_out_ref[0] = jnp.concatenate(
        [jnp.where(matched, gb_x1, neg1), jnp.where(matched, gb_y1, neg1),
         jnp.where(matched, gb_x2, neg1), jnp.where(matched, gb_y2, neg1)],
        axis=0)

    cls_lab = jnp.where(matched, g_lab, neg1).astype(jnp.int32)
    iota_c = jax.lax.broadcasted_iota(jnp.int32, (C, A), 0)
    cls_out_ref[0] = (iota_c == cls_lab).astype(f32) * norm

    dist_out_ref[0] = jnp.where(matched, g_dst, neg1) * norm
    fg_out_ref[0] = jnp.ones((1, A), f32)


def kernel(scores, decode_bboxes, distances, anchors, gt_labels, gt_bboxes,
           gt_distances, gt_mask):
    del distances
    B, A, C = scores.shape
    G = gt_labels.shape[1]

    scores_t = jnp.transpose(scores, (0, 2, 1))
    decode_t = jnp.transpose(decode_bboxes, (0, 2, 1))
    anchors_t = jnp.transpose(anchors, (1, 0))
    labels3 = gt_labels.reshape(B, G, 1)
    gdist3 = gt_distances.reshape(B, G, 1)
    gmask3 = gt_mask.astype(jnp.float32)

    out_shapes = (
        jax.ShapeDtypeStruct((B, 4, A), jnp.float32),
        jax.ShapeDtypeStruct((B, C, A), jnp.float32),
        jax.ShapeDtypeStruct((B, 1, A), jnp.float32),
        jax.ShapeDtypeStruct((B, 1, A), jnp.float32),
    )

    def row_spec(shape):
        return pl.BlockSpec((1,) + shape, lambda b: (b, 0, 0))

    bbox_l, cls_oh, dist_l, fg = pl.pallas_call(
        _encoder_kernel,
        grid=(B,),
        in_specs=[
            row_spec((C, A)),
            row_spec((4, A)),
            pl.BlockSpec((2, A), lambda b: (0, 0)),
            row_spec((G, 1)),
            row_spec((G, 4)),
            row_spec((G, 1)),
            row_spec((G, 1)),
        ],
        out_specs=(row_spec((4, A)), row_spec((C, A)),
                   row_spec((1, A)), row_spec((1, A))),
        out_shape=out_shapes,
    )(scores_t, decode_t, anchors_t, labels3, gt_bboxes, gdist3, gmask3)

    bbox_labels = jnp.transpose(bbox_l, (0, 2, 1))
    class_labels_oh = jnp.transpose(cls_oh, (0, 2, 1))
    dist_labels = dist_l.reshape(B, A)
    fg_mask = fg.reshape(B, A)
    return bbox_labels, class_labels_oh, dist_labels, fg_mask


# speedup vs baseline: 59.4709x; 1.4946x over previous
import numpy as np
import jax
import jax.numpy as jnp
from jax.experimental import pallas as pl

_NUM_CLASSES = 80
_K = 10
_EPSILON = 1e-09


def _atan_pos(x):
    t38 = 2.414213562373095
    t8 = 0.4142135623730950
    use_big = x > t38
    use_mid = (x > t8) & (~use_big)
    arg = jnp.where(use_big, -1.0 / x, jnp.where(use_mid, (x - 1.0) / (x + 1.0), x))
    base = jnp.where(use_big, jnp.float32(np.pi / 2),
                     jnp.where(use_mid, jnp.float32(np.pi / 4), jnp.float32(0.0)))
    z = arg * arg
    poly = (((8.05374449538e-2 * z - 1.38776856032e-1) * z
             + 1.99777106478e-1) * z - 3.33329491539e-1) * z * arg + arg
    return base + poly


def _encoder_kernel(scores_t_ref, decode_t_ref, anchors_t_ref, gt_labels_ref,
                    gt_bboxes_ref, gt_dist_ref, gt_mask_ref,
                    bbox_out_ref, cls_out_ref, dist_out_ref, fg_out_ref):
    f32 = jnp.float32
    C, A = scores_t_ref.shape[1], scores_t_ref.shape[2]
    G = gt_bboxes_ref.shape[1]

    scores_t = scores_t_ref[0]
    dec = decode_t_ref[0]
    anc = anchors_t_ref[...]
    gtb = gt_bboxes_ref[0]
    labels = gt_labels_ref[0]
    gmask = gt_mask_ref[0]
    gdist = gt_dist_ref[0]

    d_x1 = dec[0:1, :]
    d_y1 = dec[1:2, :]
    d_x2 = dec[2:3, :]
    d_y2 = dec[3:4, :]
    a_x = anc[0:1, :]
    a_y = anc[1:2, :]
    g_x1 = gtb[:, 0:1]
    g_y1 = gtb[:, 1:2]
    g_x2 = gtb[:, 2:3]
    g_y2 = gtb[:, 3:4]

    cls = jnp.maximum(labels, 0)
    onehot_cls = (jax.lax.broadcasted_iota(jnp.int32, (G, C), 1) == cls).astype(f32)
    bbox_scores = jax.lax.dot_general(
        onehot_cls, scores_t, (((1,), (0,)), ((), ())),
        preferred_element_type=f32, precision=jax.lax.Precision.HIGHEST)

    eps = 1e-9
    x1 = jnp.maximum(g_x1, d_x1)
    y1 = jnp.maximum(g_y1, d_y1)
    x2 = jnp.minimum(g_x2, d_x2)
    y2 = jnp.minimum(g_y2, d_y2)
    inter = jnp.maximum(x2 - x1, 0.0) * jnp.maximum(y2 - y1, 0.0)
    w1 = g_x2 - g_x1
    h1 = g_y2 - g_y1
    w2 = d_x2 - d_x1
    h2 = d_y2 - d_y1
    union = w1 * h1 + w2 * h2 - inter
    iou = inter / (union + eps)
    cw = jnp.maximum(g_x2, d_x2) - jnp.minimum(g_x1, d_x1)
    ch = jnp.maximum(g_y2, d_y2) - jnp.minimum(g_y1, d_y1)
    c2 = cw * cw + ch * ch + eps
    dx = d_x1 + d_x2 - g_x1 - g_x2
    dy = d_y1 + d_y2 - g_y1 - g_y2
    rho2 = (dx * dx + dy * dy) * 0.25
    atan_g = _atan_pos(w1 / (h1 + eps))
    atan_d = _atan_pos(w2 / (h2 + eps))
    dv = atan_d - atan_g
    v = (4.0 / (np.pi ** 2)) * (dv * dv)
    alpha_t = v / (v - iou + 1.0 + eps)
    ciou = iou - (rho2 / c2 + v * alpha_t)

    o2 = ciou * ciou
    o6 = o2 * o2 * o2
    inbox = (g_x1 < a_x) & (g_y1 < a_y) & (g_x2 > a_x) & (g_y2 > a_y)
    keep = inbox & (gmask > 0.0)
    align = jnp.where(keep, jnp.sqrt(bbox_scores) * o6, 0.0)

    work = align
    thresh = jnp.full((G, 1), -1.0, f32)
    for _ in range(_K):
        thresh = jnp.max(work, axis=1, keepdims=True)
        work = jnp.where(work == thresh, -1.0, work)
    match = ((align >= thresh) & (align > 0.0)).astype(f32)

    ov_m = ciou * match
    al_m = align * match

    best_ov = jnp.max(ov_m, axis=0, keepdims=True)
    iota_g = jax.lax.broadcasted_iota(jnp.int32, (G, A), 0)
    best_g = jnp.min(jnp.where(ov_m == best_ov, iota_g, G), axis=0, keepdims=True)
    matched = best_ov > 0.0
    onehot_a = (iota_g == best_g).astype(f32)

    gvals = jnp.concatenate(
        [gtb, gdist, jnp.zeros((G, 3), f32)], axis=1)
    gathered = jax.lax.dot_general(
        gvals, onehot_a, (((0,), (0,)), ((), ())),
        preferred_element_type=f32, precision=jax.lax.Precision.HIGHEST)

    max_al = jnp.max(al_m, axis=1, keepdims=True)
    max_ov = jnp.max(ov_m, axis=1, keepdims=True)
    ratio = max_ov / (max_al + _EPSILON)
    norm = jnp.max(al_m * ratio, axis=0, keepdims=True)

    neg1 = jnp.float32(-1.0)
    bbox_out_ref[0] = jnp.where(matched, gathered[0:4, :], neg1)

    sel_scaled = onehot_a * jnp.where(matched, norm, 0.0)
    cls_out_ref[0] = jax.lax.dot_general(
        onehot_cls, sel_scaled, (((0,), (0,)), ((), ())),
        preferred_element_type=f32, precision=jax.lax.Precision.HIGHEST)

    dist_out_ref[0] = jnp.where(matched, gathered[4:5, :], neg1) * norm
    fg_out_ref[0] = jnp.ones((1, A), f32)


def kernel(scores, decode_bboxes, distances, anchors, gt_labels, gt_bboxes,
           gt_distances, gt_mask):
    del distances
    B, A, C = scores.shape
    G = gt_labels.shape[1]

    scores_t = jnp.transpose(scores, (0, 2, 1))
    decode_t = jnp.transpose(decode_bboxes, (0, 2, 1))
    anchors_t = jnp.transpose(anchors, (1, 0))
    labels3 = gt_labels.reshape(B, G, 1)
    gdist3 = gt_distances.reshape(B, G, 1)
    gmask3 = gt_mask.astype(jnp.float32)

    out_shapes = (
        jax.ShapeDtypeStruct((B, 4, A), jnp.float32),
        jax.ShapeDtypeStruct((B, C, A), jnp.float32),
        jax.ShapeDtypeStruct((B, 1, A), jnp.float32),
        jax.ShapeDtypeStruct((B, 1, A), jnp.float32),
    )

    def row_spec(shape):
        return pl.BlockSpec((1,) + shape, lambda b: (b, 0, 0))

    bbox_l, cls_oh, dist_l, fg = pl.pallas_call(
        _encoder_kernel,
        grid=(B,),
        in_specs=[
            row_spec((C, A)),
            row_spec((4, A)),
            pl.BlockSpec((2, A), lambda b: (0, 0)),
            row_spec((G, 1)),
            row_spec((G, 4)),
            row_spec((G, 1)),
            row_spec((G, 1)),
        ],
        out_specs=(row_spec((4, A)), row_spec((C, A)),
                   row_spec((1, A)), row_spec((1, A))),
        out_shape=out_shapes,
    )(scores_t, decode_t, anchors_t, labels3, gt_bboxes, gdist3, gmask3)

    bbox_labels = jnp.transpose(bbox_l, (0, 2, 1))
    class_labels_oh = jnp.transpose(cls_oh, (0, 2, 1))
    dist_labels = dist_l.reshape(B, A)
    fg_mask = fg.reshape(B, A)
    return bbox_labels, class_labels_oh, dist_labels, fg_mask
